# baseline (speedup 1.0000x reference)
import functools

import jax
import jax.numpy as jnp
from jax import lax
from jax.experimental import pallas as pl
from jax.experimental.pallas import tpu as pltpu
from jax.experimental.pallas import tpu_sc as plsc

_NC = 2
_NS = 16
_NW = _NC * _NS
_CH = 128


@functools.lru_cache(maxsize=None)
def _build_gather(n, d):
    per_w = n // _NW
    n_chunks = per_w // _CH
    mesh = plsc.VectorSubcoreMesh(core_axis_name="c", subcore_axis_name="s")

    @functools.partial(
        pl.kernel,
        out_type=jax.ShapeDtypeStruct((n, d), jnp.float32),
        mesh=mesh,
        scratch_types=[
            pltpu.VMEM((n_chunks, _CH), jnp.int32),
            pltpu.VMEM((2, _CH, d), jnp.float32),
            pltpu.SemaphoreType.DMA((2,)),
            pltpu.SemaphoreType.DMA((2,)),
        ],
    )
    def gather_kernel(table_hbm, idx_hbm, out_hbm, idx_v, rows_v, gsem, ssem):
        wid = lax.axis_index("s") * _NC + lax.axis_index("c")
        base = wid * per_w
        pltpu.sync_copy(idx_hbm.at[wid], idx_v)
        pltpu.async_copy(table_hbm.at[idx_v.at[0]], rows_v.at[0], gsem.at[0])

        @pl.loop(0, n_chunks, step=2)
        def round_(r):
            for sub in range(2):
                c = r + sub
                slot = sub
                other = 1 - sub
                pltpu.make_async_copy(
                    table_hbm.at[idx_v.at[c]], rows_v.at[slot], gsem.at[slot]
                ).wait()

                @pl.when(c + 1 < n_chunks)
                def _():
                    @pl.when(c >= 1)
                    def _():
                        pltpu.make_async_copy(
                            rows_v.at[other],
                            out_hbm.at[pl.ds(base, _CH)],
                            ssem.at[other],
                        ).wait()

                    pltpu.async_copy(
                        table_hbm.at[idx_v.at[c + 1]], rows_v.at[other], gsem.at[other]
                    )

                pltpu.async_copy(
                    rows_v.at[slot],
                    out_hbm.at[pl.ds(base + c * _CH, _CH)],
                    ssem.at[slot],
                )

        for slot in range(2):
            pltpu.make_async_copy(
                rows_v.at[slot], out_hbm.at[pl.ds(base, _CH)], ssem.at[slot]
            ).wait()

    return gather_kernel


_GB = 8


@functools.lru_cache(maxsize=None)
def _build_fold(b, s, d):
    def fold_kernel(x_ref, out_ref):
        for j in range(_GB):
            out_ref[j] = x_ref[pl.ds(j * s, s), :]

    return pl.pallas_call(
        fold_kernel,
        grid=(b // _GB,),
        in_specs=[pl.BlockSpec((_GB * s, d), lambda i: (i, 0))],
        out_specs=pl.BlockSpec((_GB, s, d), lambda i: (i, 0, 0)),
        out_shape=jax.ShapeDtypeStruct((b, s, d), jnp.float32),
    )


def kernel(words_ids, table):
    b, s = words_ids.shape
    v, d = table.shape
    n = b * s
    idx = words_ids.reshape(_NW, n // _NW // _CH, _CH).astype(jnp.int32)
    flat = _build_gather(n, d)(table, idx)
    return _build_fold(b, s, d)(flat)


# speedup vs baseline: 2.5391x; 2.5391x over previous
import functools

import jax
import jax.numpy as jnp
from jax import lax
from jax.experimental import pallas as pl
from jax.experimental.pallas import tpu as pltpu
from jax.experimental.pallas import tpu_sc as plsc

_NC = 2
_NS = 16
_NW = _NC * _NS
_K = 1


@functools.lru_cache(maxsize=None)
def _build_gather(nb, s, d):
    pairs_per_w = nb // _NW // 2
    s2 = 2 * s
    mesh = plsc.VectorSubcoreMesh(core_axis_name="c", subcore_axis_name="s")

    @functools.partial(
        pl.kernel,
        out_type=jax.ShapeDtypeStruct((nb, s, d), jnp.float32),
        mesh=mesh,
        scratch_types=[
            pltpu.VMEM((pairs_per_w, s2), jnp.int32),
            pltpu.VMEM((2, s2, d), jnp.float32),
            pltpu.SemaphoreType.DMA((2,)),
            pltpu.SemaphoreType.DMA((2,)),
        ],
    )
    def gather_kernel(table_hbm, idx_hbm, out_hbm, idx_v, rows_v, gsem, ssem):
        wid = lax.axis_index("s") * _NC + lax.axis_index("c")
        base = wid * pairs_per_w
        pltpu.sync_copy(idx_hbm.at[pl.ds(base, pairs_per_w)], idx_v)
        pltpu.async_copy(table_hbm.at[idx_v.at[0]], rows_v.at[0], gsem.at[0])

        @pl.loop(0, pairs_per_w, step=2)
        def round_(r):
            for sub in range(2):
                c = r + sub
                slot = sub
                other = 1 - sub
                pltpu.make_async_copy(
                    table_hbm.at[idx_v.at[c]], rows_v.at[slot], gsem.at[slot]
                ).wait()

                @pl.when(c + 1 < pairs_per_w)
                def _():
                    @pl.when(c >= 1)
                    def _():
                        for h in range(2):
                            pltpu.make_async_copy(
                                rows_v.at[other].at[pl.ds(h * s, s)],
                                out_hbm.at[base],
                                ssem.at[other],
                            ).wait()

                    pltpu.async_copy(
                        table_hbm.at[idx_v.at[c + 1]], rows_v.at[other], gsem.at[other]
                    )

                for h in range(2):
                    pltpu.async_copy(
                        rows_v.at[slot].at[pl.ds(h * s, s)],
                        out_hbm.at[2 * (base + c) + h],
                        ssem.at[slot],
                    )

        for slot in range(2):
            for h in range(2):
                pltpu.make_async_copy(
                    rows_v.at[slot].at[pl.ds(h * s, s)],
                    out_hbm.at[base],
                    ssem.at[slot],
                ).wait()

    return gather_kernel


def kernel(words_ids, table):
    b, s = words_ids.shape
    v, d = table.shape
    nb = b // _K
    idx = words_ids.reshape(_K, nb // 2, 2 * s).astype(jnp.int32)
    return _build_gather(nb, s, d)(table, idx[0])
